# baseline (device time: 102487 ns/iter reference)
import jax
import jax.numpy as jnp
from jax import lax
from jax.experimental import pallas as pl
from jax.experimental.pallas import tpu as pltpu

N_DEV = 16
_GELU_C = 0.7978845608028654

_ORD = ((0, 1, 2, 3), (1, 2, 0, 3), (2, 3, 1, 0), (3, 2, 1, 0))


def _gelu(y):
    return 0.5 * y * (1.0 + jnp.tanh(_GELU_C * (y + 0.044715 * y * y * y)))


def kernel(x, w_mat):
    m_per, k = x.shape
    _, n_per = w_mat.shape
    m_half = m_per // 2

    def body(x_ref, w_ref, out_ref, comm_ref, recv_sems, z_sems,
             pr_col_sems, pl_col_sems, pr_diag_sems, pl_diag_sems):
        my = lax.axis_index("i")
        z = my // 4
        q = lax.rem(my, 4)
        base = my - q
        q_l = base + lax.rem(q + 3, 4)
        q_r = base + lax.rem(q + 1, 4)

        barrier = pltpu.get_barrier_semaphore()
        n_nbrs = 5
        for nbr in (q_l, q_r):
            pl.semaphore_signal(
                barrier, inc=1,
                device_id=(nbr,), device_id_type=pl.DeviceIdType.MESH,
            )
        for zp in range(4):
            peer = 4 * zp + q

            @pl.when(zp != z)
            def _(peer=peer):
                pl.semaphore_signal(
                    barrier, inc=1,
                    device_id=(peer,), device_id_type=pl.DeviceIdType.MESH,
                )
        pl.semaphore_wait(barrier, n_nbrs)

        comm_ref[pl.ds(2 * my, 2)] = x_ref[...].reshape(2, m_half, k)

        def slab128(p):
            out_ref[pl.ds(p * m_per, m_per), :] = _gelu(
                jnp.dot(
                    comm_ref[pl.ds(2 * p, 2)].reshape(m_per, k),
                    w_ref[...],
                    preferred_element_type=jnp.float32,
                )
            )

        def slab64(slot):
            out_ref[pl.ds(slot * m_half, m_half), :] = _gelu(
                jnp.dot(
                    comm_ref[slot],
                    w_ref[...],
                    preferred_element_type=jnp.float32,
                )
            )

        def copy_chunk(p, send_sem, dev):
            return pltpu.make_async_remote_copy(
                src_ref=comm_ref.at[pl.ds(2 * p, 2)],
                dst_ref=comm_ref.at[pl.ds(2 * p, 2)],
                send_sem=send_sem,
                recv_sem=recv_sems.at[2 * p],
                device_id=(dev,),
                device_id_type=pl.DeviceIdType.MESH,
            )

        def copy_half(slot, send_sem, dev):
            return pltpu.make_async_remote_copy(
                src_ref=comm_ref.at[slot],
                dst_ref=comm_ref.at[slot],
                send_sem=send_sem,
                recv_sem=recv_sems.at[slot],
                device_id=(dev,),
                device_id_type=pl.DeviceIdType.MESH,
            )

        for d in (1, 2, 3):
            @pl.when(z + d <= 3)
            def _(d=d):
                copy_chunk(my, z_sems.at[d - 1], my + 4 * d).start()

            @pl.when(z - d >= 0)
            def _(d=d):
                copy_chunk(my, z_sems.at[2 + d], my - 4 * d).start()

        copy_chunk(my, pr_col_sems.at[z], q_r).start()
        copy_chunk(my, pl_col_sems.at[z], q_l).start()
        slab128(my)

        for t in range(3):
            for zp_expr, cond in (
                (z + 1 + t, z + 1 + t <= 3),
                (z - 1 - t, z - 1 - t >= 0),
            ):
                @pl.when(cond)
                def _(zp=zp_expr):
                    p = 4 * zp + q
                    copy_chunk(p, z_sems.at[0], q_l).wait_recv()
                    copy_chunk(p, pr_col_sems.at[zp], q_r).start()
                    copy_chunk(p, pl_col_sems.at[zp], q_l).start()
                    slab128(p)

        def ord_r(r):
            o = jnp.int32(_ORD[3][r])
            for zz in (2, 1, 0):
                o = jnp.where(z == zz, jnp.int32(_ORD[zz][r]), o)
            return o

        for r in range(4):
            o = ord_r(r)
            p_l = 4 * o + lax.rem(q + 3, 4)
            p_r = 4 * o + lax.rem(q + 1, 4)
            copy_chunk(p_l, z_sems.at[0], q_l).wait_recv()
            copy_half(2 * p_l, pr_diag_sems.at[o], q_r).start()
            slab128(p_l)
            copy_chunk(p_r, z_sems.at[0], q_r).wait_recv()
            copy_half(2 * p_r + 1, pl_diag_sems.at[o], q_l).start()
            slab128(p_r)
        for r in range(4):
            o = ord_r(r)
            sd = 2 * (4 * o + lax.rem(q + 2, 4))
            copy_half(sd, z_sems.at[0], q_l).wait_recv()
            slab64(sd)
            copy_half(sd + 1, z_sems.at[0], q_r).wait_recv()
            slab64(sd + 1)

        for d in (1, 2, 3):
            @pl.when(z + d <= 3)
            def _(d=d):
                copy_chunk(my, z_sems.at[d - 1], q_r).wait_send()

            @pl.when(z - d >= 0)
            def _(d=d):
                copy_chunk(my, z_sems.at[2 + d], q_r).wait_send()
        for zp in range(4):
            copy_chunk(my, pr_col_sems.at[zp], q_r).wait_send()
            copy_chunk(my, pl_col_sems.at[zp], q_l).wait_send()
            copy_half(2 * my, pr_diag_sems.at[zp], q_r).wait_send()
            copy_half(2 * my, pl_diag_sems.at[zp], q_l).wait_send()

    out_shape = jax.ShapeDtypeStruct((N_DEV * m_per, n_per), jnp.float32)
    return pl.pallas_call(
        body,
        out_shape=out_shape,
        in_specs=[
            pl.BlockSpec(memory_space=pltpu.VMEM),
            pl.BlockSpec(memory_space=pltpu.VMEM),
        ],
        out_specs=pl.BlockSpec(memory_space=pltpu.VMEM),
        scratch_shapes=[
            pltpu.VMEM((2 * N_DEV, m_half, k), jnp.float32),
            pltpu.SemaphoreType.DMA((2 * N_DEV,)),
            pltpu.SemaphoreType.DMA((6,)),
            pltpu.SemaphoreType.DMA((4,)),
            pltpu.SemaphoreType.DMA((4,)),
            pltpu.SemaphoreType.DMA((4,)),
            pltpu.SemaphoreType.DMA((4,)),
        ],
        compiler_params=pltpu.CompilerParams(collective_id=0),
    )(x, w_mat)


# device time: 76912 ns/iter; 1.3325x vs baseline; 1.3325x over previous
import jax
import jax.numpy as jnp
from jax import lax
from jax.experimental import pallas as pl
from jax.experimental.pallas import tpu as pltpu

N_DEV = 16
_GELU_C = 0.7978845608028654

_ORD = ((0, 1, 2, 3), (1, 2, 0, 3), (2, 3, 1, 0), (3, 2, 1, 0))


def _gelu(y):
    return 0.5 * y * (1.0 + jnp.tanh(_GELU_C * (y + 0.044715 * y * y * y)))


def kernel(x, w_mat):
    m_per, k = x.shape
    _, n_per = w_mat.shape
    m_half = m_per // 2

    def body(x_ref, w_ref, out_ref, comm_ref, recv_sems,
             zup_sems, zdn_sems, pr_col_sems, pl_col_sems,
             pr_diag_sems, pl_diag_sems):
        my = lax.axis_index("i")
        z = my // 4
        q = lax.rem(my, 4)
        base = my - q
        q_l = base + lax.rem(q + 3, 4)
        q_r = base + lax.rem(q + 1, 4)
        up = lax.rem(my + 4, N_DEV)
        dn = lax.rem(my - 4 + N_DEV, N_DEV)
        diag = base + lax.rem(q + 2, 4)

        def col_slot(zp, h):
            return 2 * (4 * zp + q) + h

        barrier = pltpu.get_barrier_semaphore()
        for nbr in (q_l, q_r):
            pl.semaphore_signal(
                barrier, inc=1,
                device_id=(nbr,), device_id_type=pl.DeviceIdType.MESH,
            )

        @pl.when(z < 3)
        def _():
            pl.semaphore_signal(
                barrier, inc=1,
                device_id=(up,), device_id_type=pl.DeviceIdType.MESH,
            )

        @pl.when(z > 0)
        def _():
            pl.semaphore_signal(
                barrier, inc=1,
                device_id=(dn,), device_id_type=pl.DeviceIdType.MESH,
            )

        pl.semaphore_wait(barrier, 2)

        @pl.when(z < 3)
        def _():
            pl.semaphore_wait(barrier, 1)

        @pl.when(z > 0)
        def _():
            pl.semaphore_wait(barrier, 1)

        comm_ref[pl.ds(2 * my, 2)] = x_ref[...].reshape(2, m_half, k)

        def slab(slot):
            out_ref[pl.ds(slot * m_half, m_half), :] = _gelu(
                jnp.dot(
                    comm_ref[slot],
                    w_ref[...],
                    preferred_element_type=jnp.float32,
                )
            )

        def copy(slot, send_sem, dev):
            return pltpu.make_async_remote_copy(
                src_ref=comm_ref.at[slot],
                dst_ref=comm_ref.at[slot],
                send_sem=send_sem,
                recv_sem=recv_sems.at[slot],
                device_id=(dev,),
                device_id_type=pl.DeviceIdType.MESH,
            )

        def wait_recv(slot):
            copy(slot, zup_sems.at[0], q_l).wait_recv()

        for h in (0, 1):
            s = 2 * my + h

            @pl.when(z < 3)
            def _(s=s, h=h):
                copy(s, zup_sems.at[2 * z + h], up).start()

            @pl.when(z > 0)
            def _(s=s, h=h):
                copy(s, zdn_sems.at[2 * z + h], dn).start()

            copy(s, pr_col_sems.at[2 * z + h], q_r).start()
            copy(s, pl_col_sems.at[2 * z + h], q_l).start()
        slab(2 * my)
        slab(2 * my + 1)

        for t in range(3):
            for h in (0, 1):
                zp = z + 1 + t
                s = col_slot(zp, h)
                cond = zp <= 3

                @pl.when(cond)
                def _(s=s, h=h, zp=zp):
                    wait_recv(s)

                @pl.when(cond & (z > 0))
                def _(s=s, h=h, zp=zp):
                    copy(s, zdn_sems.at[2 * zp + h], dn).start()

                @pl.when(cond)
                def _(s=s, h=h, zp=zp):
                    copy(s, pr_col_sems.at[2 * zp + h], q_r).start()
                    copy(s, pl_col_sems.at[2 * zp + h], q_l).start()
                    slab(s)
            for h in (0, 1):
                zp = z - 1 - t
                s = col_slot(zp, h)
                cond = zp >= 0

                @pl.when(cond)
                def _(s=s, h=h, zp=zp):
                    wait_recv(s)

                @pl.when(cond & (z < 3))
                def _(s=s, h=h, zp=zp):
                    copy(s, zup_sems.at[2 * zp + h], up).start()

                @pl.when(cond)
                def _(s=s, h=h, zp=zp):
                    copy(s, pr_col_sems.at[2 * zp + h], q_r).start()
                    copy(s, pl_col_sems.at[2 * zp + h], q_l).start()
                    slab(s)

        def ord_r(r):
            o = jnp.int32(_ORD[3][r])
            for zz in (2, 1, 0):
                o = jnp.where(z == zz, jnp.int32(_ORD[zz][r]), o)
            return o

        for r in range(4):
            o = ord_r(r)
            sl0 = 2 * (4 * o + lax.rem(q + 3, 4))
            sr1 = 2 * (4 * o + lax.rem(q + 1, 4)) + 1
            wait_recv(sl0)
            copy(sl0, pr_diag_sems.at[o], q_r).start()
            slab(sl0)
            wait_recv(sr1)
            copy(sr1, pl_diag_sems.at[o], q_l).start()
            slab(sr1)
            wait_recv(sl0 + 1)
            slab(sl0 + 1)
            wait_recv(sr1 - 1)
            slab(sr1 - 1)
        for r in range(4):
            o = ord_r(r)
            sd = 2 * (4 * o + lax.rem(q + 2, 4))
            wait_recv(sd)
            slab(sd)
            wait_recv(sd + 1)
            slab(sd + 1)

        for zp in range(4):
            for h in (0, 1):
                i = 2 * zp + h
                copy(i, pr_col_sems.at[i], q_r).wait_send()
                copy(i, pl_col_sems.at[i], q_l).wait_send()

                @pl.when((z < 3) & (zp <= z))
                def _(i=i):
                    copy(i, zup_sems.at[i], up).wait_send()

                @pl.when((z > 0) & (zp >= z))
                def _(i=i):
                    copy(i, zdn_sems.at[i], dn).wait_send()

            copy(zp, pr_diag_sems.at[zp], q_r).wait_send()
            copy(zp, pl_diag_sems.at[zp], q_l).wait_send()

    out_shape = jax.ShapeDtypeStruct((N_DEV * m_per, n_per), jnp.float32)
    return pl.pallas_call(
        body,
        out_shape=out_shape,
        in_specs=[
            pl.BlockSpec(memory_space=pltpu.VMEM),
            pl.BlockSpec(memory_space=pltpu.VMEM),
        ],
        out_specs=pl.BlockSpec(memory_space=pltpu.VMEM),
        scratch_shapes=[
            pltpu.VMEM((2 * N_DEV, m_half, k), jnp.float32),
            pltpu.SemaphoreType.DMA((2 * N_DEV,)),
            pltpu.SemaphoreType.DMA((8,)),
            pltpu.SemaphoreType.DMA((8,)),
            pltpu.SemaphoreType.DMA((8,)),
            pltpu.SemaphoreType.DMA((8,)),
            pltpu.SemaphoreType.DMA((4,)),
            pltpu.SemaphoreType.DMA((4,)),
        ],
        compiler_params=pltpu.CompilerParams(collective_id=0),
    )(x, w_mat)
